# baseline (speedup 1.0000x reference)
import functools
import math

import jax
import jax.numpy as jnp
from jax import lax
from jax.experimental import pallas as pl
from jax.experimental.pallas import tpu as pltpu
from jax.experimental.pallas import tpu_sc as plsc

_LANES = 16
_IDX_CHUNK = 128


@functools.partial(jax.jit, static_argnums=(3, 4, 5))
def _embed_lookup(idx2d, token_table, pos_table, n_rows, seq_len, scale):
    d = token_table.shape[1]
    info = plsc.get_sparse_core_info()
    n_workers = info.num_cores * info.num_subcores
    per_w = n_rows // n_workers
    chunks = per_w // _IDX_CHUNK
    mesh = plsc.VectorSubcoreMesh(core_axis_name="c", subcore_axis_name="s")

    @functools.partial(
        pl.kernel,
        mesh=mesh,
        compiler_params=pltpu.CompilerParams(use_tc_tiling_on_sc=False),
        out_type=jax.ShapeDtypeStruct((n_rows, d), jnp.float32),
        scratch_types=[
            pltpu.VMEM((chunks, _IDX_CHUNK), jnp.int32),
            pltpu.VMEM((per_w, d), jnp.float32),
            pltpu.VMEM((per_w, d), jnp.float32),
            pltpu.SemaphoreType.DMA,
            pltpu.SemaphoreType.DMA,
        ],
    )
    def run(idx_hbm, tok_hbm, pos_hbm, out_hbm, idx_v, rows_v, pos_v, gsem, psem):
        wid = lax.axis_index("s") * info.num_cores + lax.axis_index("c")
        base = wid * per_w
        pos_base = lax.rem(base, seq_len)
        pos_copy = pltpu.async_copy(
            pos_hbm.at[pl.ds(pos_base, per_w)], pos_v, psem)
        pltpu.sync_copy(idx_hbm.at[pl.ds(wid * chunks, chunks)], idx_v)
        gathers = []
        for j in range(chunks):
            gathers.append(pltpu.async_copy(
                tok_hbm.at[idx_v.at[j]],
                rows_v.at[pl.ds(j * _IDX_CHUNK, _IDX_CHUNK)],
                gsem))
        for g in gathers:
            g.wait()
        pos_copy.wait()

        def body(i, _):
            for c in range(d // _LANES):
                sl = pl.ds(c * _LANES, _LANES)
                rows_v[i, sl] = rows_v[i, sl] * scale + pos_v[i, sl]
            return ()

        lax.fori_loop(0, per_w, body, (), unroll=2)
        pltpu.sync_copy(rows_v, out_hbm.at[pl.ds(base, per_w)])

    return run(idx2d, token_table, pos_table)


def kernel(inputs, token_table, pos_table):
    b, s = inputs.shape
    d = token_table.shape[1]
    n = b * s
    scale = float(math.sqrt(d))
    idx2d = inputs.reshape(n // _IDX_CHUNK, _IDX_CHUNK).astype(jnp.int32)
    out = _embed_lookup(idx2d, token_table, pos_table, n, s, scale)
    return out.reshape(b, s, d)


# speedup vs baseline: 1.5521x; 1.5521x over previous
import functools
import math

import jax
import jax.numpy as jnp
from jax import lax
from jax.experimental import pallas as pl
from jax.experimental.pallas import tpu as pltpu
from jax.experimental.pallas import tpu_sc as plsc

_LANES = 16
_IDX_CHUNK = 128


@functools.partial(jax.jit, static_argnums=(3, 4, 5))
def _embed_lookup(idx2d, token_table, pos_table, n_rows, seq_len, scale):
    d = token_table.shape[1]
    info = plsc.get_sparse_core_info()
    n_workers = info.num_cores * info.num_subcores
    per_w = n_rows // n_workers
    chunks = per_w // _IDX_CHUNK
    mesh = plsc.VectorSubcoreMesh(core_axis_name="c", subcore_axis_name="s")

    @functools.partial(
        pl.kernel,
        mesh=mesh,
        compiler_params=pltpu.CompilerParams(use_tc_tiling_on_sc=False),
        out_type=jax.ShapeDtypeStruct((n_rows, d), jnp.float32),
        scratch_types=[
            pltpu.VMEM((chunks, _IDX_CHUNK), jnp.int32),
            pltpu.VMEM((per_w, d), jnp.float32),
            pltpu.VMEM((per_w, d), jnp.float32),
            pltpu.SemaphoreType.DMA,
            pltpu.SemaphoreType.DMA,
        ],
    )
    def run(idx_hbm, tok_hbm, pos_hbm, out_hbm, idx_v, rows_v, pos_v, gsem, psem):
        wid = lax.axis_index("s") * info.num_cores + lax.axis_index("c")
        base = wid * per_w
        pos_base = lax.rem(base, seq_len)
        pos_copy = pltpu.async_copy(
            pos_hbm.at[pl.ds(pos_base, per_w)], pos_v, psem)
        pltpu.sync_copy(idx_hbm.at[pl.ds(wid * chunks, chunks)], idx_v)
        gathers = []
        for j in range(chunks):
            gathers.append(pltpu.async_copy(
                tok_hbm.at[idx_v.at[j]],
                rows_v.at[pl.ds(j * _IDX_CHUNK, _IDX_CHUNK)],
                gsem))
        for g in gathers:
            g.wait()
        pos_copy.wait()

        def body(i, _):
            for c in range(d // _LANES):
                sl = pl.ds(c * _LANES, _LANES)
                rows_v[i, sl] = rows_v[i, sl] * scale + pos_v[i, sl]
            return ()

        lax.fori_loop(0, per_w, body, (), unroll=2)
        pltpu.sync_copy(rows_v, out_hbm.at[pl.ds(base, per_w)])

    return run(idx2d, token_table, pos_table)


def kernel(inputs, token_table, pos_table):
    b, s = inputs.shape
    d = token_table.shape[1]
    n = b * s
    scale = float(math.sqrt(d))
    idx2d = inputs.reshape(n // _IDX_CHUNK, _IDX_CHUNK).astype(jnp.int32)
    out = _embed_lookup(idx2d, token_table, pos_table[:s], n, s, scale)
    return out.reshape(b, s, d)
